# baseline (speedup 1.0000x reference)
import functools

import jax
import jax.numpy as jnp
from jax import lax
from jax.experimental import pallas as pl
from jax.experimental.pallas import tpu as pltpu
from jax.experimental.pallas import tpu_sc as plsc

_NUM_CORES = 2
_NUM_SUBCORES = 16
_NW = _NUM_CORES * _NUM_SUBCORES

_NBUF = 6


@functools.lru_cache(maxsize=None)
def _make_gather(V, D, Bx, H, nbuf):
    bcols = Bx // _NW
    assert bcols * _NW == Bx
    mesh = plsc.VectorSubcoreMesh(core_axis_name="c", subcore_axis_name="s")

    @functools.partial(
        pl.kernel,
        out_type=jax.ShapeDtypeStruct((H * Bx, D), jnp.float32),
        mesh=mesh,
        compiler_params=pltpu.CompilerParams(use_tc_tiling_on_sc=True),
        scratch_types=(
            [pltpu.VMEM((H, bcols), jnp.int32),
             pltpu.VMEM_SHARED((V, D), jnp.float32)]
            + [pltpu.VMEM((bcols, D), jnp.float32) for _ in range(nbuf)]
            + [pltpu.SemaphoreType.DMA for _ in range(2 * nbuf)]
        ),
    )
    def gather(table_hbm, idx_hbm, out_hbm, idx_v, table_sh, *bufs):
        rows = bufs[:nbuf]
        gsem = bufs[nbuf:2 * nbuf]
        ssem = bufs[2 * nbuf:]
        sid = lax.axis_index("s")
        wid = sid * _NUM_CORES + lax.axis_index("c")
        b0 = wid * bcols

        @pl.when(sid == 0)
        def _():
            pltpu.sync_copy(table_hbm, table_sh)

        pltpu.sync_copy(idx_hbm.at[:, pl.ds(b0, bcols)], idx_v)
        plsc.subcore_barrier()

        def fire_gather(h, b):
            return pltpu.async_copy(
                table_sh.at[idx_v.at[h]], rows[b], gsem[b])

        def fire_store(h, b):
            return pltpu.async_copy(
                rows[b], out_hbm.at[pl.ds(h * Bx + b0, bcols)], ssem[b])

        gathers = [None] * nbuf
        stores = [None] * nbuf
        for j in range(min(nbuf - 1, H)):
            gathers[j] = fire_gather(j, j)
        for h in range(H):
            b = h % nbuf
            nh = h + nbuf - 1
            if nh < H:
                pb = nh % nbuf
                if stores[pb] is not None:
                    stores[pb].wait()
                gathers[pb] = fire_gather(nh, pb)
            gathers[b].wait()
            stores[b] = fire_store(h, b)
        for s in stores:
            if s is not None:
                s.wait()

    return gather


def kernel(x, table):
    Bx, H = x.shape
    V, D = table.shape
    idx = jnp.transpose(x)
    out = _make_gather(V, D, Bx, H, _NBUF)(table, idx)
    return out.reshape(H, Bx, D).transpose(1, 0, 2)


# speedup vs baseline: 1.0103x; 1.0103x over previous
import functools

import jax
import jax.numpy as jnp
from jax import lax
from jax.experimental import pallas as pl
from jax.experimental.pallas import tpu as pltpu
from jax.experimental.pallas import tpu_sc as plsc

_NUM_CORES = 2
_NUM_SUBCORES = 16
_NW = _NUM_CORES * _NUM_SUBCORES

_CHUNK = 200
_NBUF = 4


@functools.lru_cache(maxsize=None)
def _make_gather(V, D, B, chunk, nbuf):
    per_w = B // _NW
    nchunk = per_w // chunk
    assert per_w * _NW == B and nchunk * chunk == per_w
    mesh = plsc.VectorSubcoreMesh(core_axis_name="c", subcore_axis_name="s")

    @functools.partial(
        pl.kernel,
        out_type=jax.ShapeDtypeStruct((B, D), jnp.float32),
        mesh=mesh,
        compiler_params=pltpu.CompilerParams(use_tc_tiling_on_sc=True),
        scratch_types=(
            [pltpu.VMEM((per_w,), jnp.int32),
             pltpu.VMEM_SHARED((V, D), jnp.float32)]
            + [pltpu.VMEM((chunk, D), jnp.float32) for _ in range(nbuf)]
            + [pltpu.SemaphoreType.DMA for _ in range(2 * nbuf)]
        ),
    )
    def gather(table_hbm, idx_hbm, out_hbm, idx_all, table_sh, *bufs):
        rows = bufs[:nbuf]
        gsem = bufs[nbuf:2 * nbuf]
        ssem = bufs[2 * nbuf:]
        sid = lax.axis_index("s")
        wid = sid * _NUM_CORES + lax.axis_index("c")
        base = wid * per_w

        @pl.when(sid == 0)
        def _():
            pltpu.sync_copy(table_hbm, table_sh)

        pltpu.sync_copy(idx_hbm.at[pl.ds(base, per_w)], idx_all)
        plsc.subcore_barrier()

        def fire_gather(g, b):
            return pltpu.async_copy(
                table_sh.at[idx_all.at[pl.ds(g * chunk, chunk)]],
                rows[b], gsem[b])

        def fire_store(g, b):
            return pltpu.async_copy(
                rows[b], out_hbm.at[pl.ds(base + g * chunk, chunk)], ssem[b])

        gathers = [None] * nbuf
        stores = [None] * nbuf
        for j in range(min(nbuf - 1, nchunk)):
            gathers[j] = fire_gather(j, j)
        for g in range(nchunk):
            b = g % nbuf
            ng = g + nbuf - 1
            if ng < nchunk:
                pb = ng % nbuf
                if stores[pb] is not None:
                    stores[pb].wait()
                gathers[pb] = fire_gather(ng, pb)
            gathers[b].wait()
            stores[b] = fire_store(g, b)
        for s in stores:
            if s is not None:
                s.wait()

    return gather


def kernel(x, table):
    Bx, H = x.shape
    V, D = table.shape
    tot = Bx * H
    idx = jnp.transpose(x).reshape(tot)
    out = _make_gather(V, D, tot, _CHUNK, _NBUF)(table, idx)
    return out.reshape(H, Bx, D).transpose(1, 0, 2)
